# baseline (speedup 1.0000x reference)
import jax
import jax.numpy as jnp
from jax import lax
from jax.experimental import pallas as pl
from jax.experimental.pallas import tpu as pltpu

H = 1024
I = 512
E = 64
NG = 8
GS = E // NG
TOPK_GROUP = 4
TOP_K = 8
N_SHARED = 2
N_TOK = 512

B = 128
MAXG = (TOP_K * N_TOK) // B + E

NEG = -1e30


def _first_max_mask(work, axis):
    m = jnp.max(work, axis=axis, keepdims=True)
    ismax = work == m
    idx = lax.broadcasted_iota(jnp.int32, work.shape, axis)
    first = jnp.min(jnp.where(ismax, idx, jnp.int32(10**9)), axis=axis,
                    keepdims=True)
    return idx == first


def _gate_combine_T(x, wg):
    lT = lax.dot_general(wg, x, (((1,), (1,)), ((), ())),
                         preferred_element_type=jnp.float32)
    l3 = lT.reshape(NG, GS, N_TOK)
    work = l3
    sel4 = jnp.zeros(l3.shape, dtype=jnp.bool_)
    for _ in range(TOPK_GROUP):
        pick = _first_max_mask(work, 1)
        sel4 = jnp.logical_or(sel4, pick)
        work = jnp.where(pick, NEG, work)
    cand = jnp.where(sel4, l3, NEG).reshape(E, N_TOK)
    sel8 = jnp.zeros(cand.shape, dtype=jnp.bool_)
    work2 = cand
    for _ in range(TOP_K):
        pick = _first_max_mask(work2, 0)
        sel8 = jnp.logical_or(sel8, pick)
        work2 = jnp.where(pick, NEG, work2)
    wsel = jnp.where(sel8, lT, jnp.float32(0.0))
    wsum = jnp.sum(wsel, axis=0, keepdims=True) + jnp.float32(1e-20)
    return wsel / wsum


def _gate_body(x_ref, wg_ref, combT_ref, dest_ref, eid_ref, valid_ref):
    combT = _gate_combine_T(x_ref[...], wg_ref[...])
    combT_ref[...] = combT
    mask = (combT != 0.0).astype(jnp.float32)

    iota_t = lax.broadcasted_iota(jnp.int32, (N_TOK, N_TOK), 0)
    lt_incl = (iota_t <= lax.broadcasted_iota(
        jnp.int32, (N_TOK, N_TOK), 1)).astype(jnp.float32)
    rank_incl = jnp.dot(mask, lt_incl,
                        preferred_element_type=jnp.float32)
    cnt = rank_incl[:, N_TOK - 1:N_TOK]
    cnt_i = cnt.astype(jnp.int32)
    pc = ((cnt_i + (B - 1)) // B) * B
    iota_e = lax.broadcasted_iota(jnp.int32, (E, E), 0)
    lt_strict = (lax.broadcasted_iota(jnp.int32, (E, E), 1)
                 < iota_e).astype(jnp.float32)
    start = jnp.dot(lt_strict, pc.astype(jnp.float32),
                    preferred_element_type=jnp.float32)
    dest = jnp.where(mask != 0.0,
                     start + rank_incl - mask,
                     jnp.float32(-1.0))
    dest_ref[...] = dest

    nch = (pc // B).astype(jnp.float32)
    lt_incl_e = (lax.broadcasted_iota(jnp.int32, (E, E), 1)
                 <= iota_e).astype(jnp.float32)
    cumnch = jnp.dot(lt_incl_e, nch,
                     preferred_element_type=jnp.float32)
    nactive = cumnch[E - 1:E, 0:1]
    c_row = lax.broadcasted_iota(jnp.int32, (1, 128), 1).astype(jnp.float32)
    ones_e = jnp.ones((1, E), jnp.float32)
    eid_row = jnp.dot(ones_e, (cumnch <= c_row).astype(jnp.float32),
                      preferred_element_type=jnp.float32)
    maxeid = jnp.dot(ones_e,
                     (cumnch <= nactive - 1.0).astype(jnp.float32),
                     preferred_element_type=jnp.float32)
    eid_ref[...] = jnp.minimum(eid_row, maxeid).astype(jnp.int32)
    valid_ref[...] = (c_row < nactive).astype(jnp.int32)


def _mlp(x, w_gu, w_dn):
    h = jnp.dot(x, w_gu, preferred_element_type=jnp.float32)
    g = h[:, :I]
    u = h[:, I:]
    return jnp.dot(jax.nn.silu(g) * u, w_dn,
                   preferred_element_type=jnp.float32)


def _expert_body(eid_ref, valid_ref, x_ref, combT_ref, dest_ref,
                 wga_ref, wgb_ref, wda_ref, wdb_ref,
                 wsgu_ref, wsdn_ref, out_ref):
    g = pl.program_id(0)

    @pl.when(g == 0)
    def _init():
        acc = jnp.zeros((N_TOK, H), jnp.float32)
        for s in range(N_SHARED):
            acc = acc + _mlp(x_ref[...], wsgu_ref[s], wsdn_ref[s])
        out_ref[...] = acc

    HI = lax.Precision.HIGHEST

    @pl.when(valid_ref[0, g] == 1)
    def _chunk():
        e = eid_ref[0, g]
        onehot = (lax.broadcasted_iota(jnp.int32, (E, 1), 0) == e
                  ).astype(jnp.float32)
        d_col = lax.dot_general(dest_ref[...], onehot,
                                (((0,), (0,)), ((), ())), precision=HI,
                                preferred_element_type=jnp.float32)
        c_col = lax.dot_general(combT_ref[...], onehot,
                                (((0,), (0,)), ((), ())), precision=HI,
                                preferred_element_type=jnp.float32)
        slot = (lax.broadcasted_iota(jnp.int32, (N_TOK, B), 1) + B * g
                ).astype(jnp.float32)
        pt = (d_col == slot)
        ptf = pt.astype(jnp.float32)
        ptw = jnp.where(pt, c_col, jnp.float32(0.0))
        xg = lax.dot_general(ptf, x_ref[...], (((0,), (0,)), ((), ())),
                             precision=HI,
                             preferred_element_type=jnp.float32)
        ga = jnp.dot(xg, wga_ref[...], preferred_element_type=jnp.float32)
        ub = jnp.dot(xg, wgb_ref[...], preferred_element_type=jnp.float32)
        act = jax.nn.silu(ga) * ub
        ya = jnp.dot(act, wda_ref[...], preferred_element_type=jnp.float32)
        yb = jnp.dot(act, wdb_ref[...], preferred_element_type=jnp.float32)
        out_ref[:, :I] += lax.dot_general(
            ptw, ya, (((1,), (0,)), ((), ())), precision=HI,
            preferred_element_type=jnp.float32)
        out_ref[:, I:] += lax.dot_general(
            ptw, yb, (((1,), (0,)), ((), ())), precision=HI,
            preferred_element_type=jnp.float32)


def kernel(x, Wg, W_gu, W_dn, Ws_gu, Ws_dn):
    combT, dest, eid, valid = pl.pallas_call(
        _gate_body,
        in_specs=[
            pl.BlockSpec((N_TOK, H), lambda: (0, 0)),
            pl.BlockSpec((E, H), lambda: (0, 0)),
        ],
        out_specs=[
            pl.BlockSpec((E, N_TOK), lambda: (0, 0)),
            pl.BlockSpec((E, N_TOK), lambda: (0, 0)),
            pl.BlockSpec((1, 128), lambda: (0, 0)),
            pl.BlockSpec((1, 128), lambda: (0, 0)),
        ],
        out_shape=[
            jax.ShapeDtypeStruct((E, N_TOK), jnp.float32),
            jax.ShapeDtypeStruct((E, N_TOK), jnp.float32),
            jax.ShapeDtypeStruct((1, 128), jnp.int32),
            jax.ShapeDtypeStruct((1, 128), jnp.int32),
        ],
    )(x, Wg)

    expert_spec = pltpu.PrefetchScalarGridSpec(
        num_scalar_prefetch=2,
        grid=(MAXG,),
        in_specs=[
            pl.BlockSpec((N_TOK, H), lambda g, eid, valid: (0, 0)),
            pl.BlockSpec((E, N_TOK), lambda g, eid, valid: (0, 0)),
            pl.BlockSpec((E, N_TOK), lambda g, eid, valid: (0, 0)),
            pl.BlockSpec((None, H, I),
                         lambda g, eid, valid: (eid[0, g], 0, 0)),
            pl.BlockSpec((None, H, I),
                         lambda g, eid, valid: (eid[0, g], 0, 1)),
            pl.BlockSpec((None, I, I),
                         lambda g, eid, valid: (eid[0, g], 0, 0)),
            pl.BlockSpec((None, I, I),
                         lambda g, eid, valid: (eid[0, g], 0, 1)),
            pl.BlockSpec((N_SHARED, H, 2 * I),
                         lambda g, eid, valid: (0, 0, 0)),
            pl.BlockSpec((N_SHARED, I, H),
                         lambda g, eid, valid: (0, 0, 0)),
        ],
        out_specs=pl.BlockSpec((N_TOK, H), lambda g, eid, valid: (0, 0)),
    )
    return pl.pallas_call(
        _expert_body,
        grid_spec=expert_spec,
        out_shape=jax.ShapeDtypeStruct((N_TOK, H), jnp.float32),
        compiler_params=pltpu.CompilerParams(
            dimension_semantics=("arbitrary",),
        ),
    )(eid, valid, x, combT, dest, W_gu, W_gu, W_dn, W_dn, Ws_gu, Ws_dn)
